# baseline (speedup 1.0000x reference)
import jax
import jax.numpy as jnp
from jax import lax
from jax.experimental import pallas as pl
from jax.experimental.pallas import tpu as pltpu
from jax.experimental.pallas import tpu_sc as plsc

N = 10000
E = 320000
C = 128
B = 1024
OUTC = 1

NC, NS = 2, 16
NW = NC * NS
KCH = 128
TOTCH = 2544
EP = TOTCH * KCH
NF = 80
NSL = 79
NDEG = 10240
TRASH = 10000
RPT = NDEG // NW
AGG_PT = NDEG // NS
ZROWS = 32
GCH = 64
PF, PSL = 80, 79
RB = 1000

_mesh = plsc.VectorSubcoreMesh(core_axis_name="c", subcore_axis_name="s")


def _sc_pre_body(nid_h, bidx_h, st16_h, emb_h, dstc_h, ones_h, zeros_h,
                 embg_h, stg_h, deg_h,
                 idx_v, rows_v, st_v, didx2_v, one_v, zdeg_v, degsh, sem, ssem):
    cid = lax.axis_index("c")
    sid = lax.axis_index("s")
    wid = cid * NS + sid
    rbase = wid * RPT

    pltpu.sync_copy(ones_h, one_v)
    pltpu.sync_copy(zeros_h, zdeg_v)

    dbase = sid * AGG_PT

    def zs_it(i, carry):
        pltpu.sync_copy(zdeg_v, degsh.at[pl.ds(dbase + i * ZROWS, ZROWS)])
        return carry
    lax.fori_loop(0, AGG_PT // ZROWS, zs_it, 0)

    plsc.subcore_barrier()

    def deg_pipeline(nch, cbase):
        pltpu.sync_copy(dstc_h.at[pl.ds(cbase, nch)], didx2_v.at[pl.ds(0, nch)])

        def dg_it(g, carry):
            pltpu.async_copy(one_v, degsh.at[didx2_v.at[g, 0]], ssem, add=True)

            @pl.when(g >= 2)
            def _():
                pltpu.make_async_copy(ones_h, one_v, ssem).wait()
            return carry
        lax.fori_loop(0, nch, dg_it, 0)
        pltpu.make_async_copy(ones_h, one_v, ssem).wait()
        pltpu.make_async_copy(ones_h, one_v, ssem).wait()

    @pl.when(cid == 0)
    def _():
        deg_pipeline(PF, sid * PF)

    @pl.when(cid == 1)
    def _():
        deg_pipeline(PSL, NS * PF + sid * PSL)

    def it(j, carry):
        base = rbase + j * GCH
        pltpu.sync_copy(nid_h.at[pl.ds(base, GCH)], idx_v)
        pltpu.async_copy(emb_h.at[idx_v], rows_v, sem).wait()
        pltpu.sync_copy(rows_v, embg_h.at[pl.ds(base, GCH)])
        pltpu.sync_copy(bidx_h.at[pl.ds(base, GCH)], idx_v)
        pltpu.async_copy(st16_h.at[idx_v], st_v, sem).wait()
        pltpu.sync_copy(st_v, stg_h.at[pl.ds(base, GCH)])
        return carry
    lax.fori_loop(0, RPT // GCH, it, 0)

    plsc.subcore_barrier()
    pltpu.sync_copy(degsh.at[pl.ds(dbase, AGG_PT)],
                    deg_h.at[cid, pl.ds(dbase, AGG_PT)])


_sc_pre = pl.kernel(
    _sc_pre_body,
    out_type=(jax.ShapeDtypeStruct((NDEG, C), jnp.float32),
              jax.ShapeDtypeStruct((NDEG, C), jnp.float32),
              jax.ShapeDtypeStruct((NC, NDEG, C), jnp.float32)),
    mesh=_mesh,
    scratch_types=[pltpu.VMEM((GCH,), jnp.int32),
                   pltpu.VMEM((GCH, C), jnp.float32),
                   pltpu.VMEM((GCH, C), jnp.float32),
                   pltpu.VMEM((PF, 1, KCH), jnp.int32),
                   pltpu.VMEM((KCH, C), jnp.float32),
                   pltpu.VMEM((ZROWS, C), jnp.float32),
                   pltpu.VMEM_SHARED((NDEG, C), jnp.float32),
                   pltpu.SemaphoreType.DMA,
                   pltpu.SemaphoreType.DMA],
)


def _sc_agg_body(h_h, srcc_h, dstc_h, zeros_h, agg_h,
                 sia_v, sib_v, didx2_v, rows_a, rows_b, zbuf_v, aggsh, sem):
    cid = lax.axis_index("c")
    sid = lax.axis_index("s")
    rbase = sid * AGG_PT

    pltpu.sync_copy(zeros_h, zbuf_v)

    def za_it(i, carry):
        pltpu.sync_copy(zbuf_v, aggsh.at[pl.ds(rbase + i * ZROWS, ZROWS)])
        return carry
    lax.fori_loop(0, AGG_PT // ZROWS, za_it, 0)

    plsc.subcore_barrier()

    def edge_pipeline(nch, cbase):
        pltpu.sync_copy(dstc_h.at[pl.ds(cbase, nch)], didx2_v.at[pl.ds(0, nch)])
        pltpu.sync_copy(srcc_h.at[cbase], sia_v)
        pltpu.async_copy(h_h.at[sia_v.at[0]], rows_a, sem)

        def pair(p, carry):
            g0 = 2 * p

            @pl.when(g0 + 1 < nch)
            def _():
                pltpu.sync_copy(srcc_h.at[cbase + g0 + 1], sib_v)
                pltpu.async_copy(h_h.at[sib_v.at[0]], rows_b, sem)
            pltpu.make_async_copy(h_h.at[pl.ds(0, KCH)], rows_a, sem).wait()
            pltpu.sync_copy(rows_a, aggsh.at[didx2_v.at[g0, 0]], add=True)

            @pl.when(g0 + 2 < nch)
            def _():
                pltpu.sync_copy(srcc_h.at[cbase + g0 + 2], sia_v)
                pltpu.async_copy(h_h.at[sia_v.at[0]], rows_a, sem)

            @pl.when(g0 + 1 < nch)
            def _():
                pltpu.make_async_copy(h_h.at[pl.ds(0, KCH)], rows_b, sem).wait()
                pltpu.sync_copy(rows_b, aggsh.at[didx2_v.at[g0 + 1, 0]], add=True)
            return carry
        lax.fori_loop(0, (nch + 1) // 2, pair, 0)

    @pl.when(cid == 0)
    def _():
        edge_pipeline(NF, sid * NF)

    @pl.when(cid == 1)
    def _():
        edge_pipeline(NSL, NS * NF + sid * NSL)

    plsc.subcore_barrier()
    pltpu.sync_copy(aggsh.at[pl.ds(rbase, AGG_PT)],
                    agg_h.at[cid, pl.ds(rbase, AGG_PT)])


_sc_agg = pl.kernel(
    _sc_agg_body,
    out_type=jax.ShapeDtypeStruct((NC, NDEG, C), jnp.float32),
    mesh=_mesh,
    scratch_types=[pltpu.VMEM((1, KCH), jnp.int32),
                   pltpu.VMEM((1, KCH), jnp.int32),
                   pltpu.VMEM((NF, 1, KCH), jnp.int32),
                   pltpu.VMEM((KCH, C), jnp.float32),
                   pltpu.VMEM((KCH, C), jnp.float32),
                   pltpu.VMEM((ZROWS, C), jnp.float32),
                   pltpu.VMEM_SHARED((NDEG, C), jnp.float32),
                   pltpu.SemaphoreType.DMA],
)


def _tc_pre_body(x_ref, nt_ref, stg_ref, emb_ref, fr_ref, wt_ref, h0_ref):
    rel = stg_ref[...][:, 0:1] - nt_ref[...]
    pe = jnp.sin(rel * fr_ref[...])
    h0_ref[...] = (x_ref[...] + emb_ref[...]
                   + jnp.dot(pe, wt_ref[...], preferred_element_type=jnp.float32))


_tc_pre = pl.pallas_call(
    _tc_pre_body,
    out_shape=jax.ShapeDtypeStruct((N, C), jnp.float32),
    grid=(N // RB,),
    in_specs=[pl.BlockSpec((RB, C), lambda i: (i, 0)),
              pl.BlockSpec((RB, 1), lambda i: (i, 0)),
              pl.BlockSpec((RB, C), lambda i: (i, 0)),
              pl.BlockSpec((RB, C), lambda i: (i, 0)),
              pl.BlockSpec((1, C), lambda i: (0, 0)),
              pl.BlockSpec((C, C), lambda i: (0, 0))],
    out_specs=pl.BlockSpec((RB, C), lambda i: (i, 0)),
)


def _tc_layer_body(h_ref, aga_ref, agb_ref, dga_ref, dgb_ref,
                   ws_ref, wn_ref, b_ref, o_ref):
    deg = dga_ref[...][:, 0:1] + dgb_ref[...][:, 0:1]
    agg = (aga_ref[...] + agb_ref[...]) * (1.0 / jnp.maximum(deg, 1.0))
    o = (jnp.dot(h_ref[...], ws_ref[...], preferred_element_type=jnp.float32)
         + jnp.dot(agg, wn_ref[...], preferred_element_type=jnp.float32)
         + b_ref[...])
    o_ref[...] = jnp.maximum(o, 0.0)


_tc_layer = pl.pallas_call(
    _tc_layer_body,
    out_shape=jax.ShapeDtypeStruct((N, C), jnp.float32),
    grid=(N // RB,),
    in_specs=[pl.BlockSpec((RB, C), lambda i: (i, 0)),
              pl.BlockSpec((RB, C), lambda i: (i, 0)),
              pl.BlockSpec((RB, C), lambda i: (i, 0)),
              pl.BlockSpec((RB, C), lambda i: (i, 0)),
              pl.BlockSpec((RB, C), lambda i: (i, 0)),
              pl.BlockSpec((C, C), lambda i: (0, 0)),
              pl.BlockSpec((C, C), lambda i: (0, 0)),
              pl.BlockSpec((1, C), lambda i: (0, 0))],
    out_specs=pl.BlockSpec((RB, C), lambda i: (i, 0)),
)


def _tc_head_body(h_ref, aga_ref, agb_ref, dga_ref, dgb_ref,
                  ws_ref, wn_ref, b_ref, wh_ref, bh_ref, o_ref):
    deg = dga_ref[...][:, 0:1] + dgb_ref[...][:, 0:1]
    agg = (aga_ref[...] + agb_ref[...]) * (1.0 / jnp.maximum(deg, 1.0))
    h2 = (jnp.dot(h_ref[...], ws_ref[...], preferred_element_type=jnp.float32)
          + jnp.dot(agg, wn_ref[...], preferred_element_type=jnp.float32)
          + b_ref[...])
    o_ref[...] = (jnp.dot(h2, wh_ref[...], preferred_element_type=jnp.float32)
                  + bh_ref[...])


_tc_head = pl.pallas_call(
    _tc_head_body,
    out_shape=jax.ShapeDtypeStruct((B, C), jnp.float32),
    grid=(1,),
    in_specs=[pl.BlockSpec((B, C), lambda i: (0, 0)),
              pl.BlockSpec((B, C), lambda i: (0, 0)),
              pl.BlockSpec((B, C), lambda i: (0, 0)),
              pl.BlockSpec((B, C), lambda i: (0, 0)),
              pl.BlockSpec((B, C), lambda i: (0, 0)),
              pl.BlockSpec((C, C), lambda i: (0, 0)),
              pl.BlockSpec((C, C), lambda i: (0, 0)),
              pl.BlockSpec((1, C), lambda i: (0, 0)),
              pl.BlockSpec((C, C), lambda i: (0, 0)),
              pl.BlockSpec((1, C), lambda i: (0, 0))],
    out_specs=pl.BlockSpec((B, C), lambda i: (0, 0)),
)


def kernel(x, node_time, seed_time, batch_idx, n_id, edge_index,
           emb_table, W_t, W_self, W_neigh, b, W_head, b_head):
    src = edge_index[0]
    dst = edge_index[1]
    srcp = jnp.concatenate([src, jnp.zeros((EP - E,), jnp.int32)])
    dstp = jnp.concatenate([dst, jnp.full((EP - E,), TRASH, jnp.int32)])
    src3 = srcp.reshape(TOTCH, 1, KCH)
    dst2 = dstp.reshape(TOTCH, 1, KCH)
    nidp = jnp.concatenate([n_id, jnp.zeros((NDEG - N,), jnp.int32)])
    bidxp = jnp.concatenate([batch_idx, jnp.zeros((NDEG - N,), jnp.int32)])
    st16 = jnp.broadcast_to(seed_time[:, None], (B, C))
    freqs = jnp.exp(jnp.linspace(0.0, 4.0, C)).astype(jnp.float32).reshape(1, C)
    ones_c = jnp.ones((KCH, C), jnp.float32)
    zeros_c = jnp.zeros((ZROWS, C), jnp.float32)

    embg, stg, deg = _sc_pre(nidp, bidxp, st16, emb_table, dst2, ones_c, zeros_c)
    h0 = _tc_pre(x, node_time.reshape(N, 1), stg[:N], embg[:N], freqs, W_t)
    agg1 = _sc_agg(h0, src3, dst2, zeros_c)
    h1 = _tc_layer(h0, agg1[0, :N], agg1[1, :N], deg[0, :N], deg[1, :N],
                   W_self[0], W_neigh[0], b[0].reshape(1, C))
    agg2 = _sc_agg(h1, src3, dst2, zeros_c)
    wh_pad = jnp.pad(W_head, ((0, 0), (0, C - OUTC)))
    bh_pad = jnp.pad(b_head, (0, C - OUTC)).reshape(1, C)
    out = _tc_head(h1[:B], agg2[0, :B], agg2[1, :B], deg[0, :B], deg[1, :B],
                   W_self[1], W_neigh[1], b[1].reshape(1, C), wh_pad, bh_pad)
    return out[:, :OUTC]


# speedup vs baseline: 1.3240x; 1.3240x over previous
import jax
import jax.numpy as jnp
from jax import lax
from jax.experimental import pallas as pl
from jax.experimental.pallas import tpu as pltpu
from jax.experimental.pallas import tpu_sc as plsc

N = 10000
E = 320000
C = 128
B = 1024
OUTC = 1

NC, NS = 2, 16
NW = NC * NS
KCH = 128
EP = ((E + NW * KCH - 1) // (NW * KCH)) * (NW * KCH)
EPT = EP // NW
NCHUNK = EPT // KCH
NDEG = 10240
TRASH = 10000
RPT = NDEG // NW
AGG_PT = NDEG // NS
ZROWS = 64
GCH = 64
RB = 1000

_mesh = plsc.VectorSubcoreMesh(core_axis_name="c", subcore_axis_name="s")


def _sc_pre_body(nid_h, bidx_h, st16_h, emb_h, dstp_h, ones_h, zeros_h,
                 embg_h, stg_h, deg_h,
                 idx_v, rows_v, st_v, didx_v, one_v, zdeg_v, degsh, sem):
    cid = lax.axis_index("c")
    sid = lax.axis_index("s")
    wid = cid * NS + sid
    rbase = wid * RPT

    pltpu.sync_copy(ones_h, one_v)
    pltpu.sync_copy(zeros_h, zdeg_v)

    dbase = sid * AGG_PT

    def zs_it(i, carry):
        pltpu.sync_copy(zdeg_v, degsh.at[pl.ds(dbase + i * ZROWS, ZROWS)])
        return carry
    lax.fori_loop(0, AGG_PT // ZROWS, zs_it, 0)

    plsc.subcore_barrier()

    ebase = wid * EPT

    def dg_it(g, carry):
        pltpu.sync_copy(dstp_h.at[pl.ds(ebase + g * KCH, KCH)], didx_v)
        pltpu.sync_copy(one_v, degsh.at[didx_v], add=True)
        return carry
    lax.fori_loop(0, NCHUNK, dg_it, 0)

    def it(j, carry):
        base = rbase + j * GCH
        pltpu.sync_copy(nid_h.at[pl.ds(base, GCH)], idx_v)
        pltpu.async_copy(emb_h.at[idx_v], rows_v, sem).wait()
        pltpu.sync_copy(rows_v, embg_h.at[pl.ds(base, GCH)])
        pltpu.sync_copy(bidx_h.at[pl.ds(base, GCH)], idx_v)
        pltpu.async_copy(st16_h.at[idx_v], st_v, sem).wait()
        pltpu.sync_copy(st_v, stg_h.at[pl.ds(base, GCH)])
        return carry
    lax.fori_loop(0, RPT // GCH, it, 0)

    plsc.subcore_barrier()
    pltpu.sync_copy(degsh.at[pl.ds(dbase, AGG_PT)],
                    deg_h.at[cid, pl.ds(dbase, AGG_PT)])


_sc_pre = pl.kernel(
    _sc_pre_body,
    out_type=(jax.ShapeDtypeStruct((NDEG, C), jnp.float32),
              jax.ShapeDtypeStruct((NDEG, C), jnp.float32),
              jax.ShapeDtypeStruct((NC, NDEG, C), jnp.float32)),
    mesh=_mesh,
    scratch_types=[pltpu.VMEM((GCH,), jnp.int32),
                   pltpu.VMEM((GCH, C), jnp.float32),
                   pltpu.VMEM((GCH, C), jnp.float32),
                   pltpu.VMEM((KCH,), jnp.int32),
                   pltpu.VMEM((KCH, C), jnp.float32),
                   pltpu.VMEM((ZROWS, C), jnp.float32),
                   pltpu.VMEM_SHARED((NDEG, C), jnp.float32),
                   pltpu.SemaphoreType.DMA],
)


def _sc_agg_body(h_h, srcp_h, dstp_h, zeros_h, agg_h,
                 sia_v, sib_v, didx2_v, rows_a, rows_b, aggsh, sem):
    cid = lax.axis_index("c")
    sid = lax.axis_index("s")
    wid = cid * NS + sid
    rbase = sid * AGG_PT

    pltpu.sync_copy(dstp_h.at[wid], didx2_v)

    def za_it(i, carry):
        pltpu.sync_copy(zeros_h, aggsh.at[pl.ds(rbase + i * ZROWS, ZROWS)])
        return carry
    lax.fori_loop(0, AGG_PT // ZROWS, za_it, 0)

    plsc.subcore_barrier()

    ebase = wid * NCHUNK
    pltpu.sync_copy(srcp_h.at[ebase], sia_v)
    pltpu.async_copy(h_h.at[sia_v.at[0]], rows_a, sem)

    def pair(p, carry):
        g0 = 2 * p

        @pl.when(g0 + 1 < NCHUNK)
        def _():
            pltpu.sync_copy(srcp_h.at[ebase + g0 + 1], sib_v)
            pltpu.async_copy(h_h.at[sib_v.at[0]], rows_b, sem)
        pltpu.make_async_copy(h_h.at[pl.ds(0, KCH)], rows_a, sem).wait()
        pltpu.sync_copy(rows_a, aggsh.at[didx2_v.at[g0]], add=True)

        @pl.when(g0 + 2 < NCHUNK)
        def _():
            pltpu.sync_copy(srcp_h.at[ebase + g0 + 2], sia_v)
            pltpu.async_copy(h_h.at[sia_v.at[0]], rows_a, sem)

        @pl.when(g0 + 1 < NCHUNK)
        def _():
            pltpu.make_async_copy(h_h.at[pl.ds(0, KCH)], rows_b, sem).wait()
            pltpu.sync_copy(rows_b, aggsh.at[didx2_v.at[g0 + 1]], add=True)
        return carry
    lax.fori_loop(0, (NCHUNK + 1) // 2, pair, 0)

    plsc.subcore_barrier()
    pltpu.sync_copy(aggsh.at[pl.ds(rbase, AGG_PT)],
                    agg_h.at[cid, pl.ds(rbase, AGG_PT)])


_sc_agg = pl.kernel(
    _sc_agg_body,
    out_type=jax.ShapeDtypeStruct((NC, NDEG, C), jnp.float32),
    mesh=_mesh,
    scratch_types=[pltpu.VMEM((1, KCH), jnp.int32),
                   pltpu.VMEM((1, KCH), jnp.int32),
                   pltpu.VMEM((NCHUNK, KCH), jnp.int32),
                   pltpu.VMEM((KCH, C), jnp.float32),
                   pltpu.VMEM((KCH, C), jnp.float32),
                   pltpu.VMEM_SHARED((NDEG, C), jnp.float32),
                   pltpu.SemaphoreType.DMA],
)


def _tc_pre_body(x_ref, nt_ref, stg_ref, emb_ref, fr_ref, wt_ref, h0_ref):
    rel = stg_ref[...][:, 0:1] - nt_ref[...]
    pe = jnp.sin(rel * fr_ref[...])
    h0_ref[...] = (x_ref[...] + emb_ref[...]
                   + jnp.dot(pe, wt_ref[...], preferred_element_type=jnp.float32))


_tc_pre = pl.pallas_call(
    _tc_pre_body,
    out_shape=jax.ShapeDtypeStruct((N, C), jnp.float32),
    grid=(N // RB,),
    in_specs=[pl.BlockSpec((RB, C), lambda i: (i, 0)),
              pl.BlockSpec((RB, 1), lambda i: (i, 0)),
              pl.BlockSpec((RB, C), lambda i: (i, 0)),
              pl.BlockSpec((RB, C), lambda i: (i, 0)),
              pl.BlockSpec((1, C), lambda i: (0, 0)),
              pl.BlockSpec((C, C), lambda i: (0, 0))],
    out_specs=pl.BlockSpec((RB, C), lambda i: (i, 0)),
)


def _tc_layer_body(h_ref, aga_ref, agb_ref, dga_ref, dgb_ref,
                   ws_ref, wn_ref, b_ref, o_ref):
    deg = dga_ref[...][:, 0:1] + dgb_ref[...][:, 0:1]
    agg = (aga_ref[...] + agb_ref[...]) * (1.0 / jnp.maximum(deg, 1.0))
    o = (jnp.dot(h_ref[...], ws_ref[...], preferred_element_type=jnp.float32)
         + jnp.dot(agg, wn_ref[...], preferred_element_type=jnp.float32)
         + b_ref[...])
    o_ref[...] = jnp.maximum(o, 0.0)


_tc_layer = pl.pallas_call(
    _tc_layer_body,
    out_shape=jax.ShapeDtypeStruct((N, C), jnp.float32),
    grid=(N // RB,),
    in_specs=[pl.BlockSpec((RB, C), lambda i: (i, 0)),
              pl.BlockSpec((RB, C), lambda i: (i, 0)),
              pl.BlockSpec((RB, C), lambda i: (i, 0)),
              pl.BlockSpec((RB, C), lambda i: (i, 0)),
              pl.BlockSpec((RB, C), lambda i: (i, 0)),
              pl.BlockSpec((C, C), lambda i: (0, 0)),
              pl.BlockSpec((C, C), lambda i: (0, 0)),
              pl.BlockSpec((1, C), lambda i: (0, 0))],
    out_specs=pl.BlockSpec((RB, C), lambda i: (i, 0)),
)


def _tc_head_body(h_ref, aga_ref, agb_ref, dga_ref, dgb_ref,
                  ws_ref, wn_ref, b_ref, wh_ref, bh_ref, o_ref):
    deg = dga_ref[...][:, 0:1] + dgb_ref[...][:, 0:1]
    agg = (aga_ref[...] + agb_ref[...]) * (1.0 / jnp.maximum(deg, 1.0))
    h2 = (jnp.dot(h_ref[...], ws_ref[...], preferred_element_type=jnp.float32)
          + jnp.dot(agg, wn_ref[...], preferred_element_type=jnp.float32)
          + b_ref[...])
    o_ref[...] = (jnp.dot(h2, wh_ref[...], preferred_element_type=jnp.float32)
                  + bh_ref[...])


_tc_head = pl.pallas_call(
    _tc_head_body,
    out_shape=jax.ShapeDtypeStruct((B, C), jnp.float32),
    grid=(1,),
    in_specs=[pl.BlockSpec((B, C), lambda i: (0, 0)),
              pl.BlockSpec((B, C), lambda i: (0, 0)),
              pl.BlockSpec((B, C), lambda i: (0, 0)),
              pl.BlockSpec((B, C), lambda i: (0, 0)),
              pl.BlockSpec((B, C), lambda i: (0, 0)),
              pl.BlockSpec((C, C), lambda i: (0, 0)),
              pl.BlockSpec((C, C), lambda i: (0, 0)),
              pl.BlockSpec((1, C), lambda i: (0, 0)),
              pl.BlockSpec((C, C), lambda i: (0, 0)),
              pl.BlockSpec((1, C), lambda i: (0, 0))],
    out_specs=pl.BlockSpec((B, C), lambda i: (0, 0)),
)


def kernel(x, node_time, seed_time, batch_idx, n_id, edge_index,
           emb_table, W_t, W_self, W_neigh, b, W_head, b_head):
    src = edge_index[0]
    dst = edge_index[1]
    srcp = jnp.concatenate([src, jnp.zeros((EP - E,), jnp.int32)])
    dstp = jnp.concatenate([dst, jnp.full((EP - E,), TRASH, jnp.int32)])
    src3 = srcp.reshape(NW * NCHUNK, 1, KCH)
    dst3 = dstp.reshape(NW, NCHUNK, KCH)
    nidp = jnp.concatenate([n_id, jnp.zeros((NDEG - N,), jnp.int32)])
    bidxp = jnp.concatenate([batch_idx, jnp.zeros((NDEG - N,), jnp.int32)])
    st16 = jnp.broadcast_to(seed_time[:, None], (B, C))
    freqs = jnp.exp(jnp.linspace(0.0, 4.0, C)).astype(jnp.float32).reshape(1, C)
    ones_c = jnp.ones((KCH, C), jnp.float32)
    zeros_c = jnp.zeros((ZROWS, C), jnp.float32)

    embg, stg, deg = _sc_pre(nidp, bidxp, st16, emb_table, dstp, ones_c, zeros_c)
    h0 = _tc_pre(x, node_time.reshape(N, 1), stg[:N], embg[:N], freqs, W_t)
    agg1 = _sc_agg(h0, src3, dst3, zeros_c)
    h1 = _tc_layer(h0, agg1[0, :N], agg1[1, :N], deg[0, :N], deg[1, :N],
                   W_self[0], W_neigh[0], b[0].reshape(1, C))
    agg2 = _sc_agg(h1, src3, dst3, zeros_c)
    wh_pad = jnp.pad(W_head, ((0, 0), (0, C - OUTC)))
    bh_pad = jnp.pad(b_head, (0, C - OUTC)).reshape(1, C)
    out = _tc_head(h1[:B], agg2[0, :B], agg2[1, :B], deg[0, :B], deg[1, :B],
                   W_self[1], W_neigh[1], b[1].reshape(1, C), wh_pad, bh_pad)
    return out[:, :OUTC]


# speedup vs baseline: 1.3785x; 1.0412x over previous
import jax
import jax.numpy as jnp
from jax import lax
from jax.experimental import pallas as pl
from jax.experimental.pallas import tpu as pltpu
from jax.experimental.pallas import tpu_sc as plsc

N = 10000
E = 320000
C = 128
B = 1024
OUTC = 1

NC, NS = 2, 16
NW = NC * NS
KCH = 128
EP = ((E + NW * KCH - 1) // (NW * KCH)) * (NW * KCH)
EPT = EP // NW
NCHUNK = EPT // KCH
NDEG = 10240
TRASH = 10000
RPT = NDEG // NW
AGG_PT = NDEG // NS
ZROWS = 64
GCH = 64
RB = 1000

_mesh = plsc.VectorSubcoreMesh(core_axis_name="c", subcore_axis_name="s")


def _sc_pre_body(nid_h, bidx_h, st16_h, emb_h, dstp_h, ones_h, zeros_h,
                 embg_h, stg_h, deg_h,
                 idx_v, rows_v, st_v, didx_a, didx_b, one_v, zdeg_v, degsh,
                 sem, ssem):
    cid = lax.axis_index("c")
    sid = lax.axis_index("s")
    wid = cid * NS + sid
    rbase = wid * RPT

    pltpu.sync_copy(ones_h, one_v)
    pltpu.sync_copy(zeros_h, zdeg_v)

    dbase = sid * AGG_PT

    def zs_it(i, carry):
        pltpu.sync_copy(zdeg_v, degsh.at[pl.ds(dbase + i * ZROWS, ZROWS)])
        return carry
    lax.fori_loop(0, AGG_PT // ZROWS, zs_it, 0)

    plsc.subcore_barrier()

    ebase = wid * EPT
    pltpu.sync_copy(dstp_h.at[pl.ds(ebase, KCH)], didx_a)
    pltpu.async_copy(one_v, degsh.at[didx_a], ssem, add=True)

    def dg_pair(p, carry):
        g0 = 2 * p

        @pl.when(g0 + 1 < NCHUNK)
        def _():
            pltpu.sync_copy(dstp_h.at[pl.ds(ebase + (g0 + 1) * KCH, KCH)],
                            didx_b)
            pltpu.async_copy(one_v, degsh.at[didx_b], ssem, add=True)
        pltpu.make_async_copy(ones_h, one_v, ssem).wait()

        @pl.when(g0 + 2 < NCHUNK)
        def _():
            pltpu.sync_copy(dstp_h.at[pl.ds(ebase + (g0 + 2) * KCH, KCH)],
                            didx_a)
            pltpu.async_copy(one_v, degsh.at[didx_a], ssem, add=True)

        @pl.when(g0 + 1 < NCHUNK)
        def _():
            pltpu.make_async_copy(ones_h, one_v, ssem).wait()
        return carry
    lax.fori_loop(0, (NCHUNK + 1) // 2, dg_pair, 0)

    def it(j, carry):
        base = rbase + j * GCH
        pltpu.sync_copy(nid_h.at[pl.ds(base, GCH)], idx_v)
        pltpu.async_copy(emb_h.at[idx_v], rows_v, sem).wait()
        pltpu.sync_copy(rows_v, embg_h.at[pl.ds(base, GCH)])
        pltpu.sync_copy(bidx_h.at[pl.ds(base, GCH)], idx_v)
        pltpu.async_copy(st16_h.at[idx_v], st_v, sem).wait()
        pltpu.sync_copy(st_v, stg_h.at[pl.ds(base, GCH)])
        return carry
    lax.fori_loop(0, RPT // GCH, it, 0)

    plsc.subcore_barrier()
    pltpu.sync_copy(degsh.at[pl.ds(dbase, AGG_PT)],
                    deg_h.at[cid, pl.ds(dbase, AGG_PT)])


_sc_pre = pl.kernel(
    _sc_pre_body,
    out_type=(jax.ShapeDtypeStruct((NDEG, C), jnp.float32),
              jax.ShapeDtypeStruct((NDEG, C), jnp.float32),
              jax.ShapeDtypeStruct((NC, NDEG, C), jnp.float32)),
    mesh=_mesh,
    scratch_types=[pltpu.VMEM((GCH,), jnp.int32),
                   pltpu.VMEM((GCH, C), jnp.float32),
                   pltpu.VMEM((GCH, C), jnp.float32),
                   pltpu.VMEM((KCH,), jnp.int32),
                   pltpu.VMEM((KCH,), jnp.int32),
                   pltpu.VMEM((KCH, C), jnp.float32),
                   pltpu.VMEM((ZROWS, C), jnp.float32),
                   pltpu.VMEM_SHARED((NDEG, C), jnp.float32),
                   pltpu.SemaphoreType.DMA,
                   pltpu.SemaphoreType.DMA],
)


def _sc_agg_body(h_h, srcp_h, dstp_h, zeros_h, agg_h,
                 sia_v, sib_v, didx2_v, rows_a, rows_b, aggsh, sem):
    cid = lax.axis_index("c")
    sid = lax.axis_index("s")
    wid = cid * NS + sid
    rbase = sid * AGG_PT

    pltpu.sync_copy(dstp_h.at[wid], didx2_v)

    def za_it(i, carry):
        pltpu.sync_copy(zeros_h, aggsh.at[pl.ds(rbase + i * ZROWS, ZROWS)])
        return carry
    lax.fori_loop(0, AGG_PT // ZROWS, za_it, 0)

    plsc.subcore_barrier()

    ebase = wid * NCHUNK
    pltpu.sync_copy(srcp_h.at[ebase], sia_v)
    pltpu.async_copy(h_h.at[sia_v.at[0]], rows_a, sem)

    def pair(p, carry):
        g0 = 2 * p

        @pl.when(g0 + 1 < NCHUNK)
        def _():
            pltpu.sync_copy(srcp_h.at[ebase + g0 + 1], sib_v)
            pltpu.async_copy(h_h.at[sib_v.at[0]], rows_b, sem)
        pltpu.make_async_copy(h_h.at[pl.ds(0, KCH)], rows_a, sem).wait()
        pltpu.sync_copy(rows_a, aggsh.at[didx2_v.at[g0]], add=True)

        @pl.when(g0 + 2 < NCHUNK)
        def _():
            pltpu.sync_copy(srcp_h.at[ebase + g0 + 2], sia_v)
            pltpu.async_copy(h_h.at[sia_v.at[0]], rows_a, sem)

        @pl.when(g0 + 1 < NCHUNK)
        def _():
            pltpu.make_async_copy(h_h.at[pl.ds(0, KCH)], rows_b, sem).wait()
            pltpu.sync_copy(rows_b, aggsh.at[didx2_v.at[g0 + 1]], add=True)
        return carry
    lax.fori_loop(0, (NCHUNK + 1) // 2, pair, 0)

    plsc.subcore_barrier()
    pltpu.sync_copy(aggsh.at[pl.ds(rbase, AGG_PT)],
                    agg_h.at[cid, pl.ds(rbase, AGG_PT)])


_sc_agg = pl.kernel(
    _sc_agg_body,
    out_type=jax.ShapeDtypeStruct((NC, NDEG, C), jnp.float32),
    mesh=_mesh,
    scratch_types=[pltpu.VMEM((1, KCH), jnp.int32),
                   pltpu.VMEM((1, KCH), jnp.int32),
                   pltpu.VMEM((NCHUNK, KCH), jnp.int32),
                   pltpu.VMEM((KCH, C), jnp.float32),
                   pltpu.VMEM((KCH, C), jnp.float32),
                   pltpu.VMEM_SHARED((NDEG, C), jnp.float32),
                   pltpu.SemaphoreType.DMA],
)


def _tc_pre_body(x_ref, nt_ref, stg_ref, emb_ref, fr_ref, wt_ref, h0_ref):
    rel = stg_ref[...][:, 0:1] - nt_ref[...]
    pe = jnp.sin(rel * fr_ref[...])
    h0_ref[...] = (x_ref[...] + emb_ref[...]
                   + jnp.dot(pe, wt_ref[...], preferred_element_type=jnp.float32))


_tc_pre = pl.pallas_call(
    _tc_pre_body,
    out_shape=jax.ShapeDtypeStruct((N, C), jnp.float32),
    grid=(N // RB,),
    in_specs=[pl.BlockSpec((RB, C), lambda i: (i, 0)),
              pl.BlockSpec((RB, 1), lambda i: (i, 0)),
              pl.BlockSpec((RB, C), lambda i: (i, 0)),
              pl.BlockSpec((RB, C), lambda i: (i, 0)),
              pl.BlockSpec((1, C), lambda i: (0, 0)),
              pl.BlockSpec((C, C), lambda i: (0, 0))],
    out_specs=pl.BlockSpec((RB, C), lambda i: (i, 0)),
)


def _tc_layer_body(h_ref, aga_ref, agb_ref, dga_ref, dgb_ref,
                   ws_ref, wn_ref, b_ref, o_ref):
    deg = dga_ref[...][:, 0:1] + dgb_ref[...][:, 0:1]
    agg = (aga_ref[...] + agb_ref[...]) * (1.0 / jnp.maximum(deg, 1.0))
    o = (jnp.dot(h_ref[...], ws_ref[...], preferred_element_type=jnp.float32)
         + jnp.dot(agg, wn_ref[...], preferred_element_type=jnp.float32)
         + b_ref[...])
    o_ref[...] = jnp.maximum(o, 0.0)


_tc_layer = pl.pallas_call(
    _tc_layer_body,
    out_shape=jax.ShapeDtypeStruct((N, C), jnp.float32),
    grid=(N // RB,),
    in_specs=[pl.BlockSpec((RB, C), lambda i: (i, 0)),
              pl.BlockSpec((RB, C), lambda i: (i, 0)),
              pl.BlockSpec((RB, C), lambda i: (i, 0)),
              pl.BlockSpec((RB, C), lambda i: (i, 0)),
              pl.BlockSpec((RB, C), lambda i: (i, 0)),
              pl.BlockSpec((C, C), lambda i: (0, 0)),
              pl.BlockSpec((C, C), lambda i: (0, 0)),
              pl.BlockSpec((1, C), lambda i: (0, 0))],
    out_specs=pl.BlockSpec((RB, C), lambda i: (i, 0)),
)


def _tc_head_body(h_ref, aga_ref, agb_ref, dga_ref, dgb_ref,
                  ws_ref, wn_ref, b_ref, wh_ref, bh_ref, o_ref):
    deg = dga_ref[...][:, 0:1] + dgb_ref[...][:, 0:1]
    agg = (aga_ref[...] + agb_ref[...]) * (1.0 / jnp.maximum(deg, 1.0))
    h2 = (jnp.dot(h_ref[...], ws_ref[...], preferred_element_type=jnp.float32)
          + jnp.dot(agg, wn_ref[...], preferred_element_type=jnp.float32)
          + b_ref[...])
    o_ref[...] = (jnp.dot(h2, wh_ref[...], preferred_element_type=jnp.float32)
                  + bh_ref[...])


_tc_head = pl.pallas_call(
    _tc_head_body,
    out_shape=jax.ShapeDtypeStruct((B, C), jnp.float32),
    grid=(1,),
    in_specs=[pl.BlockSpec((B, C), lambda i: (0, 0)),
              pl.BlockSpec((B, C), lambda i: (0, 0)),
              pl.BlockSpec((B, C), lambda i: (0, 0)),
              pl.BlockSpec((B, C), lambda i: (0, 0)),
              pl.BlockSpec((B, C), lambda i: (0, 0)),
              pl.BlockSpec((C, C), lambda i: (0, 0)),
              pl.BlockSpec((C, C), lambda i: (0, 0)),
              pl.BlockSpec((1, C), lambda i: (0, 0)),
              pl.BlockSpec((C, C), lambda i: (0, 0)),
              pl.BlockSpec((1, C), lambda i: (0, 0))],
    out_specs=pl.BlockSpec((B, C), lambda i: (0, 0)),
)


def kernel(x, node_time, seed_time, batch_idx, n_id, edge_index,
           emb_table, W_t, W_self, W_neigh, b, W_head, b_head):
    src = edge_index[0]
    dst = edge_index[1]
    srcp = jnp.concatenate([src, jnp.zeros((EP - E,), jnp.int32)])
    dstp = jnp.concatenate([dst, jnp.full((EP - E,), TRASH, jnp.int32)])
    src3 = srcp.reshape(NW * NCHUNK, 1, KCH)
    dst3 = dstp.reshape(NW, NCHUNK, KCH)
    nidp = jnp.concatenate([n_id, jnp.zeros((NDEG - N,), jnp.int32)])
    bidxp = jnp.concatenate([batch_idx, jnp.zeros((NDEG - N,), jnp.int32)])
    st16 = jnp.broadcast_to(seed_time[:, None], (B, C))
    freqs = jnp.exp(jnp.linspace(0.0, 4.0, C)).astype(jnp.float32).reshape(1, C)
    ones_c = jnp.ones((KCH, C), jnp.float32)
    zeros_c = jnp.zeros((ZROWS, C), jnp.float32)

    embg, stg, deg = _sc_pre(nidp, bidxp, st16, emb_table, dstp, ones_c, zeros_c)
    h0 = _tc_pre(x, node_time.reshape(N, 1), stg[:N], embg[:N], freqs, W_t)
    agg1 = _sc_agg(h0, src3, dst3, zeros_c)
    h1 = _tc_layer(h0, agg1[0, :N], agg1[1, :N], deg[0, :N], deg[1, :N],
                   W_self[0], W_neigh[0], b[0].reshape(1, C))
    agg2 = _sc_agg(h1, src3, dst3, zeros_c)
    wh_pad = jnp.pad(W_head, ((0, 0), (0, C - OUTC)))
    bh_pad = jnp.pad(b_head, (0, C - OUTC)).reshape(1, C)
    out = _tc_head(h1[:B], agg2[0, :B], agg2[1, :B], deg[0, :B], deg[1, :B],
                   W_self[1], W_neigh[1], b[1].reshape(1, C), wh_pad, bh_pad)
    return out[:, :OUTC]


# speedup vs baseline: 1.6226x; 1.1770x over previous
import jax
import jax.numpy as jnp
from jax import lax
from jax.experimental import pallas as pl
from jax.experimental.pallas import tpu as pltpu
from jax.experimental.pallas import tpu_sc as plsc

N = 10000
E = 320000
C = 128
B = 1024
OUTC = 1

NC, NS = 2, 16
NW = NC * NS
KCH = 128
EP = ((E + NW * KCH - 1) // (NW * KCH)) * (NW * KCH)
EPT = EP // NW
NCHUNK = EPT // KCH
NDEG = 10240
TRASH = 10000
RPT = NDEG // NW
AGG_PT = NDEG // NS
ZROWS = 64
GCH = 64
RB = 1000

_mesh = plsc.VectorSubcoreMesh(core_axis_name="c", subcore_axis_name="s")


def _sc_pre_body(nid_h, bidx_h, st16_h, emb_h, dstp_h, ones_h, zeros_h,
                 embg_h, stg_h, deg_h,
                 idx_v, idx2_v, rows_v, st_v, didx_a, didx_b, one_v, zdeg_v,
                 degsh, sem, ssem):
    cid = lax.axis_index("c")
    sid = lax.axis_index("s")
    wid = cid * NS + sid
    rbase = wid * RPT

    pltpu.sync_copy(ones_h, one_v)
    pltpu.sync_copy(zeros_h, zdeg_v)

    dbase = sid * AGG_PT

    def zs_it(i, carry):
        pltpu.sync_copy(zdeg_v, degsh.at[pl.ds(dbase + i * ZROWS, ZROWS)])
        return carry
    lax.fori_loop(0, AGG_PT // ZROWS, zs_it, 0)

    plsc.subcore_barrier()

    ebase = wid * EPT
    pltpu.sync_copy(dstp_h.at[pl.ds(ebase, KCH)], didx_a)
    pltpu.async_copy(one_v, degsh.at[didx_a], ssem, add=True)

    def dg_pair(p, carry):
        g0 = 2 * p

        @pl.when(g0 + 1 < NCHUNK)
        def _():
            pltpu.sync_copy(dstp_h.at[pl.ds(ebase + (g0 + 1) * KCH, KCH)],
                            didx_b)
            pltpu.async_copy(one_v, degsh.at[didx_b], ssem, add=True)
        pltpu.make_async_copy(ones_h, one_v, ssem).wait()

        @pl.when(g0 + 2 < NCHUNK)
        def _():
            pltpu.sync_copy(dstp_h.at[pl.ds(ebase + (g0 + 2) * KCH, KCH)],
                            didx_a)
            pltpu.async_copy(one_v, degsh.at[didx_a], ssem, add=True)

        @pl.when(g0 + 1 < NCHUNK)
        def _():
            pltpu.make_async_copy(ones_h, one_v, ssem).wait()
        return carry
    lax.fori_loop(0, (NCHUNK + 1) // 2, dg_pair, 0)

    def it(j, carry):
        base = rbase + j * GCH
        pltpu.sync_copy(nid_h.at[pl.ds(base, GCH)], idx_v)
        pltpu.async_copy(emb_h.at[idx_v], rows_v, sem)
        pltpu.sync_copy(bidx_h.at[pl.ds(base, GCH)], idx2_v)
        pltpu.async_copy(st16_h.at[idx2_v], st_v, sem)
        pltpu.make_async_copy(emb_h.at[pl.ds(0, GCH)], rows_v, sem).wait()
        pltpu.sync_copy(rows_v, embg_h.at[pl.ds(base, GCH)])
        pltpu.make_async_copy(st16_h.at[pl.ds(0, GCH)], st_v, sem).wait()
        pltpu.sync_copy(st_v, stg_h.at[pl.ds(base, GCH)])
        return carry
    lax.fori_loop(0, RPT // GCH, it, 0)

    plsc.subcore_barrier()
    pltpu.sync_copy(degsh.at[pl.ds(dbase, AGG_PT)],
                    deg_h.at[cid, pl.ds(dbase, AGG_PT)])


_sc_pre = pl.kernel(
    _sc_pre_body,
    out_type=(jax.ShapeDtypeStruct((NDEG, C), jnp.float32),
              jax.ShapeDtypeStruct((NDEG, C), jnp.float32),
              jax.ShapeDtypeStruct((NC, NDEG, C), jnp.float32)),
    mesh=_mesh,
    scratch_types=[pltpu.VMEM((GCH,), jnp.int32),
                   pltpu.VMEM((GCH,), jnp.int32),
                   pltpu.VMEM((GCH, C), jnp.float32),
                   pltpu.VMEM((GCH, C), jnp.float32),
                   pltpu.VMEM((KCH,), jnp.int32),
                   pltpu.VMEM((KCH,), jnp.int32),
                   pltpu.VMEM((KCH, C), jnp.float32),
                   pltpu.VMEM((ZROWS, C), jnp.float32),
                   pltpu.VMEM_SHARED((NDEG, C), jnp.float32),
                   pltpu.SemaphoreType.DMA,
                   pltpu.SemaphoreType.DMA],
)


def _sc_agg_body(h_h, srcp_h, dstp_h, zeros_h, agg_h,
                 sia_v, sib_v, didx2_v, rows_a, rows_b, aggsh, sem):
    cid = lax.axis_index("c")
    sid = lax.axis_index("s")
    wid = cid * NS + sid
    rbase = sid * AGG_PT

    pltpu.sync_copy(dstp_h.at[wid], didx2_v)

    def za_it(i, carry):
        pltpu.sync_copy(zeros_h, aggsh.at[pl.ds(rbase + i * ZROWS, ZROWS)])
        return carry
    lax.fori_loop(0, AGG_PT // ZROWS, za_it, 0)

    plsc.subcore_barrier()

    ebase = wid * NCHUNK
    pltpu.sync_copy(srcp_h.at[ebase], sia_v)
    pltpu.async_copy(h_h.at[sia_v.at[0]], rows_a, sem)

    def pair(p, carry):
        g0 = 2 * p

        @pl.when(g0 + 1 < NCHUNK)
        def _():
            pltpu.sync_copy(srcp_h.at[ebase + g0 + 1], sib_v)
            pltpu.async_copy(h_h.at[sib_v.at[0]], rows_b, sem)
        pltpu.make_async_copy(h_h.at[pl.ds(0, KCH)], rows_a, sem).wait()
        pltpu.sync_copy(rows_a, aggsh.at[didx2_v.at[g0]], add=True)

        @pl.when(g0 + 2 < NCHUNK)
        def _():
            pltpu.sync_copy(srcp_h.at[ebase + g0 + 2], sia_v)
            pltpu.async_copy(h_h.at[sia_v.at[0]], rows_a, sem)

        @pl.when(g0 + 1 < NCHUNK)
        def _():
            pltpu.make_async_copy(h_h.at[pl.ds(0, KCH)], rows_b, sem).wait()
            pltpu.sync_copy(rows_b, aggsh.at[didx2_v.at[g0 + 1]], add=True)
        return carry
    lax.fori_loop(0, (NCHUNK + 1) // 2, pair, 0)

    plsc.subcore_barrier()
    pltpu.sync_copy(aggsh.at[pl.ds(rbase, AGG_PT)],
                    agg_h.at[cid, pl.ds(rbase, AGG_PT)])


_sc_agg = pl.kernel(
    _sc_agg_body,
    out_type=jax.ShapeDtypeStruct((NC, NDEG, C), jnp.float32),
    mesh=_mesh,
    scratch_types=[pltpu.VMEM((1, KCH), jnp.int32),
                   pltpu.VMEM((1, KCH), jnp.int32),
                   pltpu.VMEM((NCHUNK, KCH), jnp.int32),
                   pltpu.VMEM((KCH, C), jnp.float32),
                   pltpu.VMEM((KCH, C), jnp.float32),
                   pltpu.VMEM_SHARED((NDEG, C), jnp.float32),
                   pltpu.SemaphoreType.DMA],
)


def _tc_pre_body(x_ref, nt_ref, stg_ref, emb_ref, fr_ref, wt_ref, h0_ref):
    rel = stg_ref[...][:, 0:1] - nt_ref[...]
    pe = jnp.sin(rel * fr_ref[...])
    h0_ref[...] = (x_ref[...] + emb_ref[...]
                   + jnp.dot(pe, wt_ref[...], preferred_element_type=jnp.float32))


_tc_pre = pl.pallas_call(
    _tc_pre_body,
    out_shape=jax.ShapeDtypeStruct((N, C), jnp.float32),
    grid=(N // RB,),
    in_specs=[pl.BlockSpec((RB, C), lambda i: (i, 0)),
              pl.BlockSpec((RB, 1), lambda i: (i, 0)),
              pl.BlockSpec((RB, C), lambda i: (i, 0)),
              pl.BlockSpec((RB, C), lambda i: (i, 0)),
              pl.BlockSpec((1, C), lambda i: (0, 0)),
              pl.BlockSpec((C, C), lambda i: (0, 0))],
    out_specs=pl.BlockSpec((RB, C), lambda i: (i, 0)),
)


def _tc_layer_body(h_ref, aga_ref, agb_ref, dga_ref, dgb_ref,
                   ws_ref, wn_ref, b_ref, o_ref):
    deg = dga_ref[...][:, 0:1] + dgb_ref[...][:, 0:1]
    agg = (aga_ref[...] + agb_ref[...]) * (1.0 / jnp.maximum(deg, 1.0))
    o = (jnp.dot(h_ref[...], ws_ref[...], preferred_element_type=jnp.float32)
         + jnp.dot(agg, wn_ref[...], preferred_element_type=jnp.float32)
         + b_ref[...])
    o_ref[...] = jnp.maximum(o, 0.0)


_tc_layer = pl.pallas_call(
    _tc_layer_body,
    out_shape=jax.ShapeDtypeStruct((N, C), jnp.float32),
    grid=(N // RB,),
    in_specs=[pl.BlockSpec((RB, C), lambda i: (i, 0)),
              pl.BlockSpec((RB, C), lambda i: (i, 0)),
              pl.BlockSpec((RB, C), lambda i: (i, 0)),
              pl.BlockSpec((RB, C), lambda i: (i, 0)),
              pl.BlockSpec((RB, C), lambda i: (i, 0)),
              pl.BlockSpec((C, C), lambda i: (0, 0)),
              pl.BlockSpec((C, C), lambda i: (0, 0)),
              pl.BlockSpec((1, C), lambda i: (0, 0))],
    out_specs=pl.BlockSpec((RB, C), lambda i: (i, 0)),
)


def _tc_head_body(h_ref, aga_ref, agb_ref, dga_ref, dgb_ref,
                  ws_ref, wn_ref, b_ref, wh_ref, bh_ref, o_ref):
    deg = dga_ref[...][:, 0:1] + dgb_ref[...][:, 0:1]
    agg = (aga_ref[...] + agb_ref[...]) * (1.0 / jnp.maximum(deg, 1.0))
    h2 = (jnp.dot(h_ref[...], ws_ref[...], preferred_element_type=jnp.float32)
          + jnp.dot(agg, wn_ref[...], preferred_element_type=jnp.float32)
          + b_ref[...])
    o_ref[...] = (jnp.dot(h2, wh_ref[...], preferred_element_type=jnp.float32)
                  + bh_ref[...])


_tc_head = pl.pallas_call(
    _tc_head_body,
    out_shape=jax.ShapeDtypeStruct((B, C), jnp.float32),
    grid=(1,),
    in_specs=[pl.BlockSpec((B, C), lambda i: (0, 0)),
              pl.BlockSpec((B, C), lambda i: (0, 0)),
              pl.BlockSpec((B, C), lambda i: (0, 0)),
              pl.BlockSpec((B, C), lambda i: (0, 0)),
              pl.BlockSpec((B, C), lambda i: (0, 0)),
              pl.BlockSpec((C, C), lambda i: (0, 0)),
              pl.BlockSpec((C, C), lambda i: (0, 0)),
              pl.BlockSpec((1, C), lambda i: (0, 0)),
              pl.BlockSpec((C, C), lambda i: (0, 0)),
              pl.BlockSpec((1, C), lambda i: (0, 0))],
    out_specs=pl.BlockSpec((B, C), lambda i: (0, 0)),
)


def kernel(x, node_time, seed_time, batch_idx, n_id, edge_index,
           emb_table, W_t, W_self, W_neigh, b, W_head, b_head):
    src = edge_index[0]
    dst = edge_index[1]
    srcp = jnp.concatenate([src, jnp.zeros((EP - E,), jnp.int32)])
    dstp = jnp.concatenate([dst, jnp.full((EP - E,), TRASH, jnp.int32)])
    src3 = srcp.reshape(NW * NCHUNK, 1, KCH)
    dst3 = dstp.reshape(NW, NCHUNK, KCH)
    nidp = jnp.concatenate([n_id, jnp.zeros((NDEG - N,), jnp.int32)])
    bidxp = jnp.concatenate([batch_idx, jnp.zeros((NDEG - N,), jnp.int32)])
    st16 = jnp.broadcast_to(seed_time[:, None], (B, C))
    freqs = jnp.exp(jnp.linspace(0.0, 4.0, C)).astype(jnp.float32).reshape(1, C)
    ones_c = jnp.ones((KCH, C), jnp.float32)
    zeros_c = jnp.zeros((ZROWS, C), jnp.float32)

    embg, stg, deg = _sc_pre(nidp, bidxp, st16, emb_table, dstp, ones_c, zeros_c)
    h0 = _tc_pre(x, node_time.reshape(N, 1), stg[:N], embg[:N], freqs, W_t)
    agg1 = _sc_agg(h0, src3, dst3, zeros_c)
    h1 = _tc_layer(h0, agg1[0, :N], agg1[1, :N], deg[0, :N], deg[1, :N],
                   W_self[0], W_neigh[0], b[0].reshape(1, C))
    agg2 = _sc_agg(h1, src3, dst3, zeros_c)
    wh_pad = jnp.pad(W_head, ((0, 0), (0, C - OUTC)))
    bh_pad = jnp.pad(b_head, (0, C - OUTC)).reshape(1, C)
    out = _tc_head(h1[:B], agg2[0, :B], agg2[1, :B], deg[0, :B], deg[1, :B],
                   W_self[1], W_neigh[1], b[1].reshape(1, C), wh_pad, bh_pad)
    return out[:, :OUTC]
